# baseline (speedup 1.0000x reference)
import functools

import jax
import jax.numpy as jnp
from jax import lax
from jax.experimental import pallas as pl
from jax.experimental.pallas import tpu as pltpu
from jax.experimental.pallas import tpu_sc as plsc

B = 4096
D = 256
K = 8192
TEMP = 0.05
BR = 512
COLT = 2048
NBLK = B // BR
LOG2E = 1.4426950408889634


def _sc_gather(targets, feats):
    info = plsc.get_sparse_core_info()
    nw = info.num_cores * info.num_subcores
    bpw = B // nw
    mesh = plsc.VectorSubcoreMesh(core_axis_name="c", subcore_axis_name="s")

    @functools.partial(
        pl.kernel, mesh=mesh,
        out_type=(jax.ShapeDtypeStruct((B, D), jnp.float32),
                  jax.ShapeDtypeStruct((B, D), jnp.float32)),
        scratch_types=[
            pltpu.VMEM((bpw,), jnp.int32),
            pltpu.VMEM((bpw,), jnp.int32),
            pltpu.VMEM((bpw, D), jnp.float32),
            pltpu.VMEM((bpw, D), jnp.float32),
            pltpu.SemaphoreType.DMA,
        ],
    )
    def k(tgt_hbm, feats_hbm, outm_hbm, outh_hbm, idx_v, idx2_v,
          rows_m, rows_h, sem):
        wid = lax.axis_index("s") * info.num_cores + lax.axis_index("c")
        base = wid * bpw
        pltpu.sync_copy(tgt_hbm.at[pl.ds(base, bpw)], idx_v)
        for j in range(bpw // 16):
            sl = pl.ds(j * 16, 16)
            idx2_v[sl] = idx_v[sl] + K
        pltpu.async_copy(feats_hbm.at[idx_v], rows_m, sem).wait()
        pltpu.async_copy(feats_hbm.at[idx2_v], rows_h, sem).wait()
        pltpu.sync_copy(rows_m, outm_hbm.at[pl.ds(base, bpw)])
        pltpu.sync_copy(rows_h, outh_hbm.at[pl.ds(base, bpw)])

    return k(targets, feats)


NCH = 2 * K // COLT


def _lse_body(x_ref, feats_ref, lse_ref, fb_ref, fstage_ref, sem_ref):
    i = pl.program_id(0)
    x = x_ref[...]
    norm = jnp.sqrt(jnp.sum(x * x, axis=1, keepdims=True))
    xn = x / jnp.maximum(norm, 1e-12)
    xnb = (xn * (LOG2E / TEMP)).astype(jnp.bfloat16)

    def tile_term(f_tile):
        l = lax.dot_general(xnb, f_tile, (((1,), (1,)), ((), ())),
                            preferred_element_type=jnp.float32)
        return jnp.sum(jnp.exp2(l), axis=1)

    def chunk_copy(c, b):
        return pltpu.make_async_copy(
            feats_ref.at[pl.ds(c * COLT, COLT)], fstage_ref.at[b],
            sem_ref.at[b])

    acc_m = jnp.zeros((BR,), jnp.float32)
    acc_h = jnp.zeros((BR,), jnp.float32)

    @pl.when(i == 0)
    def _first():
        am, ah = acc_m, acc_h
        chunk_copy(0, 0).start()
        chunk_copy(1, 1).start()
        for c in range(NCH):
            b = c % 2
            chunk_copy(c, b).wait()
            f_tile = fstage_ref[b].astype(jnp.bfloat16)
            fb_ref[pl.ds(c * COLT, COLT), :] = f_tile
            if c + 2 < NCH:
                chunk_copy(c + 2, b).start()
            s = tile_term(f_tile)
            if c < NCH // 2:
                am = am + s
            else:
                ah = ah + s
        lse_ref[:, 0] = jnp.log(am)
        lse_ref[:, 1] = jnp.log(ah)

    @pl.when(i > 0)
    def _rest():
        am, ah = acc_m, acc_h
        for c in range(NCH):
            s = tile_term(fb_ref[pl.ds(c * COLT, COLT), :])
            if c < NCH // 2:
                am = am + s
            else:
                ah = ah + s
        lse_ref[:, 0] = jnp.log(am)
        lse_ref[:, 1] = jnp.log(ah)


def _lse_call(x, feats, interpret=False):
    return pl.pallas_call(
        _lse_body,
        grid=(NBLK,),
        in_specs=[
            pl.BlockSpec((BR, D), lambda i: (i, 0)),
            pl.BlockSpec(memory_space=pltpu.HBM),
        ],
        out_specs=pl.BlockSpec((BR, 2), lambda i: (i, 0)),
        out_shape=jax.ShapeDtypeStruct((B, 2), jnp.float32),
        scratch_shapes=[
            pltpu.VMEM((2 * K, D), jnp.bfloat16),
            pltpu.VMEM((2, COLT, D), jnp.float32),
            pltpu.SemaphoreType.DMA((2,)),
        ],
        interpret=interpret,
    )(x, feats)


def _combine_body(x_ref, gm_ref, gh_ref, lse_ref, out_ref):
    x = x_ref[...]
    norm = jnp.sqrt(jnp.sum(x * x, axis=1, keepdims=True))
    xn = x / jnp.maximum(norm, 1e-12)
    t_m = jnp.sum(xn * gm_ref[...], axis=1) * (1.0 / TEMP)
    t_h = jnp.sum(xn * gh_ref[...], axis=1) * (1.0 / TEMP)
    total = jnp.sum((lse_ref[:, 0] - t_m) + (lse_ref[:, 1] - t_h))
    out_ref[0, 0] = total * (0.5 / B)


def _combine_call(x, g_m, g_h, lse, interpret=False):
    return pl.pallas_call(
        _combine_body,
        out_specs=pl.BlockSpec(memory_space=pltpu.SMEM),
        out_shape=jax.ShapeDtypeStruct((1, 1), jnp.float32),
        interpret=interpret,
    )(x, g_m, g_h, lse)


def kernel(inputs, targets, features):
    tgt = targets.astype(jnp.int32)
    g_m, g_h = _sc_gather(tgt, features)
    lse = _lse_call(inputs, features)
    out = _combine_call(inputs, g_m, g_h, lse)
    return out[0, 0]


# speedup vs baseline: 1.0430x; 1.0430x over previous
import functools

import jax
import jax.numpy as jnp
from jax import lax
from jax.experimental import pallas as pl
from jax.experimental.pallas import tpu as pltpu
from jax.experimental.pallas import tpu_sc as plsc

B = 4096
D = 256
K = 8192
TEMP = 0.05
BR = 512
COLT = 2048
NBLK = B // BR
LOG2E = 1.4426950408889634


def _sc_gather(targets, feats):
    info = plsc.get_sparse_core_info()
    nw = info.num_cores * info.num_subcores
    bpw = B // nw
    mesh = plsc.VectorSubcoreMesh(core_axis_name="c", subcore_axis_name="s")

    @functools.partial(
        pl.kernel, mesh=mesh,
        out_type=(jax.ShapeDtypeStruct((B, D), jnp.float32),
                  jax.ShapeDtypeStruct((B, D), jnp.float32)),
        scratch_types=[
            pltpu.VMEM((bpw,), jnp.int32),
            pltpu.VMEM((bpw,), jnp.int32),
            pltpu.VMEM((bpw, D), jnp.float32),
            pltpu.VMEM((bpw, D), jnp.float32),
            pltpu.SemaphoreType.DMA,
        ],
    )
    def k(tgt_hbm, feats_hbm, outm_hbm, outh_hbm, idx_v, idx2_v,
          rows_m, rows_h, sem):
        wid = lax.axis_index("s") * info.num_cores + lax.axis_index("c")
        base = wid * bpw
        pltpu.sync_copy(tgt_hbm.at[pl.ds(base, bpw)], idx_v)
        for j in range(bpw // 16):
            sl = pl.ds(j * 16, 16)
            idx2_v[sl] = idx_v[sl] + K
        pltpu.async_copy(feats_hbm.at[idx_v], rows_m, sem).wait()
        pltpu.async_copy(feats_hbm.at[idx2_v], rows_h, sem).wait()
        pltpu.sync_copy(rows_m, outm_hbm.at[pl.ds(base, bpw)])
        pltpu.sync_copy(rows_h, outh_hbm.at[pl.ds(base, bpw)])

    return k(targets, feats)


def _lse_body(x_ref, feats_ref, lse_ref, fb_ref):
    i = pl.program_id(0)

    @pl.when(i == 0)
    def _cast():
        fb_ref[...] = feats_ref[...].astype(jnp.bfloat16)

    x = x_ref[...]
    norm = jnp.sqrt(jnp.sum(x * x, axis=1, keepdims=True))
    xn = x / jnp.maximum(norm, 1e-12)
    xnb = (xn * (LOG2E / TEMP)).astype(jnp.bfloat16)

    def tile_term(f_tile):
        l = lax.dot_general(xnb, f_tile, (((1,), (1,)), ((), ())),
                            preferred_element_type=jnp.float32)
        p = jnp.exp2(l.astype(jnp.bfloat16))
        return jnp.sum(p, axis=1).astype(jnp.float32)

    acc_m = jnp.zeros((BR,), jnp.float32)
    acc_h = jnp.zeros((BR,), jnp.float32)
    for c in range(K // COLT):
        acc_m = acc_m + tile_term(fb_ref[pl.ds(c * COLT, COLT), :])
        acc_h = acc_h + tile_term(fb_ref[pl.ds(K + c * COLT, COLT), :])
    lse_ref[:, 0] = jnp.log(acc_m)
    lse_ref[:, 1] = jnp.log(acc_h)


def _lse_call(x, feats, interpret=False):
    return pl.pallas_call(
        _lse_body,
        grid=(NBLK,),
        in_specs=[
            pl.BlockSpec((BR, D), lambda i: (i, 0)),
            pl.BlockSpec((2 * K, D), lambda i: (0, 0)),
        ],
        out_specs=pl.BlockSpec((BR, 2), lambda i: (i, 0)),
        out_shape=jax.ShapeDtypeStruct((B, 2), jnp.float32),
        scratch_shapes=[pltpu.VMEM((2 * K, D), jnp.bfloat16)],
        interpret=interpret,
    )(x, feats)


def _combine_body(x_ref, gm_ref, gh_ref, lse_ref, out_ref):
    x = x_ref[...]
    norm = jnp.sqrt(jnp.sum(x * x, axis=1, keepdims=True))
    xn = x / jnp.maximum(norm, 1e-12)
    t_m = jnp.sum(xn * gm_ref[...], axis=1) * (1.0 / TEMP)
    t_h = jnp.sum(xn * gh_ref[...], axis=1) * (1.0 / TEMP)
    total = jnp.sum((lse_ref[:, 0] - t_m) + (lse_ref[:, 1] - t_h))
    out_ref[0, 0] = total * (0.5 / B)


def _combine_call(x, g_m, g_h, lse, interpret=False):
    return pl.pallas_call(
        _combine_body,
        out_specs=pl.BlockSpec(memory_space=pltpu.SMEM),
        out_shape=jax.ShapeDtypeStruct((1, 1), jnp.float32),
        interpret=interpret,
    )(x, g_m, g_h, lse)


def kernel(inputs, targets, features):
    tgt = targets.astype(jnp.int32)
    g_m, g_h = _sc_gather(tgt, features)
    lse = _lse_call(inputs, features)
    out = _combine_call(inputs, g_m, g_h, lse)
    return out[0, 0]


# speedup vs baseline: 1.0813x; 1.0368x over previous
import functools

import jax
import jax.numpy as jnp
from jax import lax
from jax.experimental import pallas as pl
from jax.experimental.pallas import tpu as pltpu
from jax.experimental.pallas import tpu_sc as plsc

B = 4096
D = 256
K = 8192
TEMP = 0.05
BR = 512
COLT = 2048
NBLK = B // BR
LOG2E = 1.4426950408889634
L = 16


def _sc_target_dots(targets, x, feats):
    info = plsc.get_sparse_core_info()
    nw = info.num_cores * info.num_subcores
    bpw = B // nw
    nch = D // L
    mesh = plsc.VectorSubcoreMesh(core_axis_name="c", subcore_axis_name="s")

    @functools.partial(
        pl.kernel, mesh=mesh,
        out_type=(jax.ShapeDtypeStruct((B, L), jnp.float32),
                  jax.ShapeDtypeStruct((B, L), jnp.float32)),
        scratch_types=[
            pltpu.VMEM((bpw,), jnp.int32),
            pltpu.VMEM((bpw,), jnp.int32),
            pltpu.VMEM((bpw, D), jnp.float32),
            pltpu.VMEM((bpw, D), jnp.float32),
            pltpu.VMEM((bpw, L), jnp.float32),
            pltpu.VMEM((bpw, L), jnp.float32),
            pltpu.SemaphoreType.DMA,
        ],
    )
    def k(tgt_hbm, x_hbm, feats_hbm, pm_hbm, ph_hbm,
          idx_v, idx2_v, rows, xv, pm_v, ph_v, sem):
        wid = lax.axis_index("s") * info.num_cores + lax.axis_index("c")
        base = wid * bpw
        pltpu.sync_copy(tgt_hbm.at[pl.ds(base, bpw)], idx_v)
        for j in range(bpw // L):
            sl = pl.ds(j * L, L)
            idx2_v[sl] = idx_v[sl] + K
        gm = pltpu.async_copy(feats_hbm.at[idx_v], rows, sem)
        pltpu.sync_copy(x_hbm.at[pl.ds(base, bpw)], xv)
        gm.wait()

        def dot_rows(p_v):
            def row_pair(r2, carry):
                for rr in range(2):
                    r = r2 * 2 + rr
                    acc = xv[r, pl.ds(0, L)] * rows[r, pl.ds(0, L)]
                    for c in range(1, nch):
                        sl = pl.ds(c * L, L)
                        acc = acc + xv[r, sl] * rows[r, sl]
                    p_v[r, :] = acc
                return carry
            lax.fori_loop(0, bpw // 2, row_pair, 0)

        dot_rows(pm_v)
        pltpu.async_copy(feats_hbm.at[idx2_v], rows, sem).wait()
        dot_rows(ph_v)
        pltpu.sync_copy(pm_v, pm_hbm.at[pl.ds(base, bpw)])
        pltpu.sync_copy(ph_v, ph_hbm.at[pl.ds(base, bpw)])

    return k(targets, x, feats)


def _lse_body(x_ref, feats_ref, lse_ref, fb_ref):
    i = pl.program_id(0)

    @pl.when(i == 0)
    def _cast():
        fb_ref[...] = feats_ref[...].astype(jnp.bfloat16)

    x = x_ref[...]
    norm = jnp.sqrt(jnp.sum(x * x, axis=1, keepdims=True))
    xn = x / jnp.maximum(norm, 1e-12)
    xnb = (xn * (LOG2E / TEMP)).astype(jnp.bfloat16)

    def tile_term(f_tile):
        l = lax.dot_general(xnb, f_tile, (((1,), (1,)), ((), ())),
                            preferred_element_type=jnp.float32)
        return jnp.sum(jnp.exp2(l), axis=1)

    acc_m = jnp.zeros((BR,), jnp.float32)
    acc_h = jnp.zeros((BR,), jnp.float32)
    for c in range(K // COLT):
        acc_m = acc_m + tile_term(fb_ref[pl.ds(c * COLT, COLT), :])
        acc_h = acc_h + tile_term(fb_ref[pl.ds(K + c * COLT, COLT), :])
    lse_ref[:, 0] = jnp.log(acc_m)
    lse_ref[:, 1] = jnp.log(acc_h)
    lse_ref[:, 2] = norm[:, 0]
    lse_ref[:, 3] = norm[:, 0]


def _lse_call(x, feats, interpret=False):
    return pl.pallas_call(
        _lse_body,
        grid=(NBLK,),
        in_specs=[
            pl.BlockSpec((BR, D), lambda i: (i, 0)),
            pl.BlockSpec((2 * K, D), lambda i: (0, 0)),
        ],
        out_specs=pl.BlockSpec((BR, 4), lambda i: (i, 0)),
        out_shape=jax.ShapeDtypeStruct((B, 4), jnp.float32),
        scratch_shapes=[pltpu.VMEM((2 * K, D), jnp.bfloat16)],
        interpret=interpret,
    )(x, feats)


def _combine_body(pm_ref, ph_ref, lse_ref, out_ref):
    inv = 1.0 / (TEMP * jnp.maximum(lse_ref[:, 2], 1e-12))
    t_m = jnp.sum(pm_ref[...], axis=1) * inv
    t_h = jnp.sum(ph_ref[...], axis=1) * inv
    total = jnp.sum((lse_ref[:, 0] - t_m) + (lse_ref[:, 1] - t_h))
    out_ref[0, 0] = total * (0.5 / B)


def _combine_call(pm, ph, lse, interpret=False):
    return pl.pallas_call(
        _combine_body,
        out_specs=pl.BlockSpec(memory_space=pltpu.SMEM),
        out_shape=jax.ShapeDtypeStruct((1, 1), jnp.float32),
        interpret=interpret,
    )(pm, ph, lse)


def kernel(inputs, targets, features):
    tgt = targets.astype(jnp.int32)
    pm, ph = _sc_target_dots(tgt, inputs, features)
    lse = _lse_call(inputs, features)
    out = _combine_call(pm, ph, lse)
    return out[0, 0]
